# Initial kernel scaffold; baseline (speedup 1.0000x reference)
import functools

import jax
import jax.numpy as jnp
from jax import lax
from jax.experimental import pallas as pl
from jax.experimental.pallas import tpu as pltpu
from jax.experimental.pallas import tpu_sc as plsc

B = 16384
L = 200
D = 32
NW = 32
BPW = B // NW
CB = 4
SEG = 100
SEGS_PER_BAG = L // SEG
SEGS_PER_CHUNK = CB * SEGS_PER_BAG
NCHUNKS = BPW // CB

_mesh = plsc.VectorSubcoreMesh(core_axis_name="c", subcore_axis_name="s")


@functools.partial(
    pl.kernel,
    mesh=_mesh,
    out_type=jax.ShapeDtypeStruct((B, D), jnp.float32),
    scratch_types=[
        pltpu.VMEM((SEGS_PER_CHUNK, SEG), jnp.int32),
        pltpu.VMEM((CB * L, D), jnp.float32),
        pltpu.VMEM((BPW, D), jnp.float32),
        pltpu.SemaphoreType.DMA,
    ],
)
def _embbag(idx_hbm, table_hbm, out_hbm, idx_v, rows_v, out_v, sem):
    wid = lax.axis_index("s") * 2 + lax.axis_index("c")
    bag0 = wid * BPW
    scale = jnp.full((16,), 1.0 / L, jnp.float32)

    def chunk_body(c, _):
        row0 = (bag0 + c * CB) * SEGS_PER_BAG
        pltpu.sync_copy(idx_hbm.at[pl.ds(row0, SEGS_PER_CHUNK)], idx_v)
        copies = []
        for s in range(SEGS_PER_CHUNK):
            copies.append(
                pltpu.async_copy(
                    table_hbm.at[idx_v.at[s]],
                    rows_v.at[pl.ds(s * SEG, SEG)],
                    sem,
                )
            )
        for cp in copies:
            cp.wait()

        for k in range(CB):
            base = k * L

            def red(j, accs):
                a0, a1 = accs
                r = base + j * 8
                for u in range(8):
                    a0 = a0 + rows_v[r + u, 0:16]
                    a1 = a1 + rows_v[r + u, 16:32]
                return (a0, a1)

            z = jnp.zeros((16,), jnp.float32)
            a0, a1 = lax.fori_loop(0, L // 8, red, (z, z))
            slot = c * CB + k
            out_v[slot, 0:16] = a0 * scale
            out_v[slot, 16:32] = a1 * scale
        return 0

    lax.fori_loop(0, NCHUNKS, chunk_body, 0)
    pltpu.sync_copy(out_v, out_hbm.at[pl.ds(bag0, BPW)])


def kernel(inlets, weight):
    idx2 = inlets.reshape(B * L // SEG, SEG)
    return _embbag(idx2, weight)


# speedup vs baseline: 12.3867x; 12.3867x over previous
import functools

import jax
import jax.numpy as jnp
from jax import lax
from jax.experimental import pallas as pl
from jax.experimental.pallas import tpu as pltpu
from jax.experimental.pallas import tpu_sc as plsc

B = 16384
L = 200
D = 32
NW = 32
BPW = B // NW
CB = 4
SEG = 100
SEGS_PER_BAG = L // SEG
SEGS_PER_CHUNK = CB * SEGS_PER_BAG
NCHUNKS = BPW // CB

_mesh = plsc.VectorSubcoreMesh(core_axis_name="c", subcore_axis_name="s")


@functools.partial(
    pl.kernel,
    mesh=_mesh,
    out_type=jax.ShapeDtypeStruct((B, D), jnp.float32),
    scratch_types=[
        pltpu.VMEM((SEGS_PER_CHUNK, SEG), jnp.int32),
        pltpu.VMEM((CB * L, D), jnp.float32),
        pltpu.VMEM((BPW, D), jnp.float32),
        pltpu.SemaphoreType.DMA,
    ],
    compiler_params=pltpu.CompilerParams(use_tc_tiling_on_sc=False),
)
def _embbag(idx_hbm, table_hbm, out_hbm, idx_v, rows_v, out_v, sem):
    wid = lax.axis_index("s") * 2 + lax.axis_index("c")
    bag0 = wid * BPW
    scale = jnp.full((16,), 1.0 / L, jnp.float32)

    def chunk_body(c, _):
        row0 = (bag0 + c * CB) * SEGS_PER_BAG
        pltpu.sync_copy(idx_hbm.at[pl.ds(row0, SEGS_PER_CHUNK)], idx_v)
        copies = []
        for s in range(SEGS_PER_CHUNK):
            copies.append(
                pltpu.async_copy(
                    table_hbm.at[idx_v.at[s]],
                    rows_v.at[pl.ds(s * SEG, SEG)],
                    sem,
                )
            )
        for cp in copies:
            cp.wait()

        for k in range(CB):
            base = k * L

            def red(j, accs):
                a0, a1 = accs
                r = base + j * 8
                for u in range(8):
                    a0 = a0 + rows_v[r + u, 0:16]
                    a1 = a1 + rows_v[r + u, 16:32]
                return (a0, a1)

            z = jnp.zeros((16,), jnp.float32)
            a0, a1 = lax.fori_loop(0, L // 8, red, (z, z))
            slot = c * CB + k
            out_v[slot, 0:16] = a0 * scale
            out_v[slot, 16:32] = a1 * scale
        return 0

    lax.fori_loop(0, NCHUNKS, chunk_body, 0)
    pltpu.sync_copy(out_v, out_hbm.at[pl.ds(bag0, BPW)])


def kernel(inlets, weight):
    idx2 = inlets.reshape(B * L // SEG, SEG)
    return _embbag(idx2, weight)


# speedup vs baseline: 15.3759x; 1.2413x over previous
import functools

import jax
import jax.numpy as jnp
from jax import lax
from jax.experimental import pallas as pl
from jax.experimental.pallas import tpu as pltpu
from jax.experimental.pallas import tpu_sc as plsc

B = 16384
L = 200
D = 32
NW = 32
BPW = B // NW
CB = 4
SEG = 100
SEGS_PER_BAG = L // SEG
SEGS_PER_CHUNK = CB * SEGS_PER_BAG
NCHUNKS = BPW // CB

_mesh = plsc.VectorSubcoreMesh(core_axis_name="c", subcore_axis_name="s")


@functools.partial(
    pl.kernel,
    mesh=_mesh,
    out_type=jax.ShapeDtypeStruct((B, D), jnp.float32),
    scratch_types=[
        pltpu.VMEM((2, SEGS_PER_CHUNK, SEG), jnp.int32),
        pltpu.VMEM((2, CB * L, D), jnp.float32),
        pltpu.VMEM((BPW, D), jnp.float32),
        pltpu.SemaphoreType.DMA((2,)),
    ],
    compiler_params=pltpu.CompilerParams(use_tc_tiling_on_sc=False),
)
def _embbag(idx_hbm, table_hbm, out_hbm, idx_v, rows_v, out_v, sems):
    wid = lax.axis_index("s") * 2 + lax.axis_index("c")
    bag0 = wid * BPW
    scale = jnp.full((16,), 1.0 / L, jnp.float32)

    def fire(c, p):
        row0 = (bag0 + c * CB) * SEGS_PER_BAG
        pltpu.sync_copy(idx_hbm.at[pl.ds(row0, SEGS_PER_CHUNK)], idx_v.at[p])
        for s in range(SEGS_PER_CHUNK):
            pltpu.async_copy(
                table_hbm.at[idx_v.at[p, s]],
                rows_v.at[p, pl.ds(s * SEG, SEG)],
                sems.at[p],
            )

    def wait(p):
        pltpu.make_async_copy(
            table_hbm.at[pl.ds(0, CB * L)], rows_v.at[p], sems.at[p]
        ).wait()

    def reduce(c, p):
        for k in range(CB):
            base = k * L

            def red(j, accs):
                acc = list(accs)
                r = base + j * 8
                for u in range(8):
                    acc[u % 4] = acc[u % 4] + rows_v[p, r + u, 0:16]
                    acc[4 + u % 4] = acc[4 + u % 4] + rows_v[p, r + u, 16:32]
                return tuple(acc)

            z = jnp.zeros((16,), jnp.float32)
            accs = lax.fori_loop(0, L // 8, red, (z,) * 8)
            a_lo = (accs[0] + accs[1]) + (accs[2] + accs[3])
            a_hi = (accs[4] + accs[5]) + (accs[6] + accs[7])
            slot = c * CB + k
            out_v[slot, 0:16] = a_lo * scale
            out_v[slot, 16:32] = a_hi * scale

    fire(0, 0)

    def body(g, _):
        c0 = g * 2
        fire(c0 + 1, 1)
        wait(0)
        reduce(c0, 0)
        fire(c0 + 2, 0)
        wait(1)
        reduce(c0 + 1, 1)
        return 0

    lax.fori_loop(0, NCHUNKS // 2 - 1, body, 0)
    c0 = NCHUNKS - 2
    fire(c0 + 1, 1)
    wait(0)
    reduce(c0, 0)
    wait(1)
    reduce(c0 + 1, 1)

    pltpu.sync_copy(out_v, out_hbm.at[pl.ds(bag0, BPW)])


def kernel(inlets, weight):
    idx2 = inlets.reshape(B * L // SEG, SEG)
    return _embbag(idx2, weight)
